# baseline (speedup 1.0000x reference)
import jax
import jax.numpy as jnp
from jax import lax
from jax.experimental import pallas as pl
from jax.experimental.pallas import tpu as pltpu
from jax.experimental.pallas import tpu_sc as plsc

VOCAB = 1000000
HIDDEN = 16
BATCH = 4096
SEQ = 200

_INFO = plsc.get_sparse_core_info()
NC = _INFO.num_cores
NS = _INFO.num_subcores
NW = NC * NS

TOTAL = BATCH * SEQ
PER_W = TOTAL // NW
CHUNK = 128
G = 10
GROUPS = PER_W // (G * CHUNK)
NBUF = 2


def _body(idx_hbm, table_hbm, out_hbm, idx_v, rows_v, sem0, sem1):
  wid = lax.axis_index("s") * NC + lax.axis_index("c")
  sems = [sem0, sem1]

  pltpu.sync_copy(idx_hbm.at[wid], idx_v)

  def fire(grp, slot):
    for g in range(G):
      pltpu.async_copy(
          table_hbm.at[idx_v.at[grp, g]], rows_v.at[slot, g], sems[slot]
      )

  def drain(slot):
    for g in range(G):
      pltpu.make_async_copy(
          table_hbm.at[idx_v.at[0, g]], rows_v.at[slot, g], sems[slot]
      ).wait()

  for b in range(NBUF):
    fire(b, b)

  @pl.loop(0, GROUPS - NBUF, step=NBUF)
  def _(grp0):
    for b in range(NBUF):
      grp = grp0 + b
      drain(b)
      pltpu.sync_copy(rows_v.at[b], out_hbm.at[wid, grp])
      fire(grp + NBUF, b)

  for b in range(NBUF):
    drain(b)
    pltpu.sync_copy(rows_v.at[b], out_hbm.at[wid, GROUPS - NBUF + b])


@jax.jit
def kernel(input_ids, word_embeddings):
  idx = input_ids.reshape(NW, GROUPS, G, CHUNK).astype(jnp.int32)
  out = pl.kernel(
      _body,
      out_type=jax.ShapeDtypeStruct((NW, GROUPS, G, CHUNK, HIDDEN),
                                    jnp.float32),
      mesh=plsc.VectorSubcoreMesh(core_axis_name="c", subcore_axis_name="s"),
      compiler_params=pltpu.CompilerParams(use_tc_tiling_on_sc=False),
      scratch_types=[
          pltpu.VMEM((GROUPS, G, CHUNK), jnp.int32),
          pltpu.VMEM((NBUF, G, CHUNK, HIDDEN), jnp.float32),
          pltpu.SemaphoreType.DMA,
          pltpu.SemaphoreType.DMA,
      ],
  )(idx, word_embeddings)
  return out.reshape(BATCH, SEQ, HIDDEN)


# speedup vs baseline: 1.5159x; 1.5159x over previous
import jax
import jax.numpy as jnp
from jax import lax
from jax.experimental import pallas as pl
from jax.experimental.pallas import tpu as pltpu
from jax.experimental.pallas import tpu_sc as plsc

VOCAB = 1000000
HIDDEN = 16
BATCH = 4096
SEQ = 200

_INFO = plsc.get_sparse_core_info()
NC = _INFO.num_cores
NS = _INFO.num_subcores
NW = NC * NS

SB = SEQ // 8
NBUF = 2

NTC = (VOCAB + 127) // 128
SLAB = 8
NSLAB = 976
TAIL0 = NSLAB * SLAB


def _transform_body(wt_hbm, tail_hbm, out_hbm, vbuf, tbuf,
                    isem0, isem1, osem0, osem1):
  wid = lax.axis_index("s") * NC + lax.axis_index("c")
  isems = [isem0, isem1]
  osems = [osem0, osem1]
  nk = (NSLAB - wid + NW - 1) // NW

  def in_cp(k, slot):
    s = wid + k * NW
    return pltpu.make_async_copy(
        wt_hbm.at[:, pl.ds(s * 1024, 1024)], vbuf.at[slot], isems[slot])

  def out_cp(k, slot):
    s = wid + k * NW
    return pltpu.make_async_copy(
        tbuf.at[slot], out_hbm.at[pl.ds(s * 128, 128)], osems[slot])

  def transpose(slot):
    @pl.loop(0, 128)
    def _(i):
      for t in range(SLAB):
        col = jnp.full((16,), i * 8 + t, jnp.int32)
        v = plsc.load_gather(vbuf, [jnp.full((16,), slot, jnp.int32),
                                    jnp.arange(16, dtype=jnp.int32), col])
        tbuf[slot, i, pl.ds(t * 16, 16)] = v

  in_cp(0, 0).start()

  @pl.loop(0, 16)
  def _(k0):
    for b in range(NBUF):
      k = k0 * 2 + b

      @pl.when(k < nk)
      def _():
        in_cp(k, b).wait()

        @pl.when(k + 1 < nk)
        def _():
          in_cp(k + 1, b ^ 1).start()

        @pl.when(k >= 2)
        def _():
          out_cp(k, b).wait()

        transpose(b)
        out_cp(k, b).start()

  out_cp(0, 0).wait()
  out_cp(0, 1).wait()

  @pl.when(wid == 0)
  def _():
    pltpu.sync_copy(tail_hbm, tbuf.at[0, pl.ds(0, 72)])
    pltpu.sync_copy(tbuf.at[0, pl.ds(0, 72)], out_hbm.at[pl.ds(124928, 72)])


def _gather_body(idx_hbm, table_hbm, out_hbm, idx_v, rows_v, trans_v,
                 gsem0, gsem1, wsem0, wsem1):
  wid = lax.axis_index("s") * NC + lax.axis_index("c")
  gsems = [gsem0, gsem1]
  wsems = [wsem0, wsem1]

  @pl.loop(0, SB)
  def _(a):
    pltpu.sync_copy(idx_hbm.at[a, wid], idx_v.at[a])

  def fire(a, slot):
    for r in range(8):
      pltpu.async_copy(table_hbm.at[idx_v.at[a, r]], rows_v.at[slot, r],
                       gsems[slot])

  def drain_g(slot):
    for r in range(8):
      pltpu.make_async_copy(table_hbm.at[idx_v.at[0, r]],
                            rows_v.at[slot, r], gsems[slot]).wait()

  def transpose(slot):
    for r in range(8):
      @pl.loop(0, HIDDEN)
      def _(h):
        h_vec = jnp.full((16,), h, jnp.int32)
        for l0 in range(8):
          v = plsc.load_gather(
              rows_v,
              [jnp.full((16,), slot, jnp.int32),
               jnp.full((16,), r, jnp.int32),
               jnp.arange(16, dtype=jnp.int32) + (l0 * 16), h_vec])
          trans_v[slot, r, h, pl.ds(l0 * 16, 16)] = v

  def write(a, slot):
    for r in range(8):
      for ht in range(2):
        pltpu.async_copy(trans_v.at[slot, r, pl.ds(ht * 8, 8)],
                         out_hbm.at[a * 8 + r, ht, wid], wsems[slot])

  def drain_w(slot):
    for r in range(8):
      for ht in range(2):
        pltpu.make_async_copy(trans_v.at[slot, r, pl.ds(ht * 8, 8)],
                              out_hbm.at[r, ht, wid], wsems[slot]).wait()

  fire(0, 0)
  fire(1, 1)

  @pl.loop(0, SB - 1, step=NBUF)
  def _(a0):
    for b in range(NBUF):
      a = a0 + b
      drain_g(b)

      @pl.when(a >= NBUF)
      def _():
        drain_w(b)

      transpose(b)

      @pl.when(a + NBUF < SB)
      def _():
        fire(a + NBUF, b)

      write(a, b)

  drain_g(0)
  drain_w(0)
  transpose(0)
  write(SB - 1, 0)
  drain_w(1)
  drain_w(0)


def kernel(input_ids, word_embeddings):
  wt = word_embeddings.T
  table_packed = pl.kernel(
      _transform_body,
      out_type=jax.ShapeDtypeStruct((125000, 128), jnp.float32),
      mesh=plsc.VectorSubcoreMesh(core_axis_name="c", subcore_axis_name="s"),
      compiler_params=pltpu.CompilerParams(use_tc_tiling_on_sc=True,
                                           needs_layout_passes=False),
      scratch_types=[
          pltpu.VMEM((NBUF, HIDDEN, 1024), jnp.float32),
          pltpu.VMEM((NBUF, 128, 128), jnp.float32),
          pltpu.SemaphoreType.DMA,
          pltpu.SemaphoreType.DMA,
          pltpu.SemaphoreType.DMA,
          pltpu.SemaphoreType.DMA,
      ],
  )(wt, word_embeddings[7808 * 128:].reshape(72, 128))

  idx = input_ids.astype(jnp.int32).reshape(32, 128, 25, 8).transpose(
      2, 0, 3, 1)
  out_p = pl.kernel(
      _gather_body,
      out_type=jax.ShapeDtypeStruct((SEQ, 2, NW, 8, 128), jnp.float32),
      mesh=plsc.VectorSubcoreMesh(core_axis_name="c", subcore_axis_name="s"),
      compiler_params=pltpu.CompilerParams(use_tc_tiling_on_sc=False,
                                           needs_layout_passes=False),
      scratch_types=[
          pltpu.VMEM((SB, 8, 128), jnp.int32),
          pltpu.VMEM((NBUF, 8, 128, HIDDEN), jnp.float32),
          pltpu.VMEM((NBUF, 8, HIDDEN, 128), jnp.float32),
          pltpu.SemaphoreType.DMA,
          pltpu.SemaphoreType.DMA,
          pltpu.SemaphoreType.DMA,
          pltpu.SemaphoreType.DMA,
      ],
  )(idx, table_packed.reshape(VOCAB, HIDDEN))
  return out_p.transpose(2, 4, 0, 1, 3).reshape(BATCH, SEQ, HIDDEN)
